# baseline (speedup 1.0000x reference)
import jax
import jax.numpy as jnp
from jax.experimental import pallas as pl
from jax.experimental.pallas import tpu as pltpu

N = 100000
D = 128
K = 16
OUT = 256
TN = 16000
NT = (N + TN - 1) // TN
TAIL = N - (NT - 1) * TN


def _body(x_ref, evt_ref, evals_ref, mask_ref, w1t_ref, b1_ref, w2t_ref,
          b2_ref, wp_ref, bp_ref, out_ref, acc_ref, m_ref):
    p = pl.program_id(0)
    j = pl.program_id(1)

    @pl.when(jnp.logical_and(p == 0, j == 0))
    def _():
        acc_ref[...] = jax.lax.dot_general(
            evt_ref[:, pl.ds(0, TN)], x_ref[...],
            dimension_numbers=(((1,), (0,)), ((), ())),
            preferred_element_type=jnp.float32)

    @pl.when(jnp.logical_and(p == 0,
                             jnp.logical_and(j > 0, j < NT - 1)))
    def _():
        evt = evt_ref[:, pl.ds(j * TN, TN)]
        acc_ref[...] += jax.lax.dot_general(
            evt, x_ref[...],
            dimension_numbers=(((1,), (0,)), ((), ())),
            preferred_element_type=jnp.float32)

    @pl.when(jnp.logical_and(p == 0, j == NT - 1))
    def _():
        evt = evt_ref[:, pl.ds((NT - 1) * TN, TAIL)]
        xfreq = acc_ref[...] + jax.lax.dot_general(
            evt, x_ref[0:TAIL, :],
            dimension_numbers=(((1,), (0,)), ((), ())),
            preferred_element_type=jnp.float32)
        h = jnp.maximum(
            jnp.dot(w1t_ref[...], evals_ref[...],
                    preferred_element_type=jnp.float32) + b1_ref[...], 0.0)
        filt = jnp.tanh(
            jnp.dot(w2t_ref[...], h,
                    preferred_element_type=jnp.float32) + b2_ref[...])
        filt = filt * mask_ref[...]
        m_ref[...] = jnp.dot(filt * xfreq, wp_ref[...],
                             preferred_element_type=jnp.float32)

    @pl.when(jnp.logical_and(p == 1, j < NT - 1))
    def _():
        evt = evt_ref[:, pl.ds(j * TN, TN)]
        out_ref[...] = jax.lax.dot_general(
            evt, m_ref[...],
            dimension_numbers=(((0,), (0,)), ((), ())),
            preferred_element_type=jnp.float32) + bp_ref[...]

    @pl.when(jnp.logical_and(p == 1, j == NT - 1))
    def _():
        evt = evt_ref[:, pl.ds((NT - 1) * TN, TAIL)]
        out_ref[0:TAIL, :] = jax.lax.dot_general(
            evt, m_ref[...],
            dimension_numbers=(((0,), (0,)), ((), ())),
            preferred_element_type=jnp.float32) + bp_ref[...]


def kernel(x, eigvecs, eigvals, eig_mask, W1, b1, W2, b2, Wp, bp):
    evt = eigvecs.T
    evals_col = eigvals.reshape(K, 1)
    mask_col = eig_mask.astype(jnp.float32).reshape(K, 1)
    w1t = W1.T
    b1_col = b1.reshape(K // 2, 1)
    w2t = W2.T
    b2_col = b2.reshape(K, 1)
    bp_row = bp.reshape(1, OUT)

    out = pl.pallas_call(
        _body,
        grid=(2, NT),
        in_specs=[
            pl.BlockSpec((TN, D), lambda p, j: ((1 - p) * j + p * (NT - 1), 0)),
            pl.BlockSpec((K, N), lambda p, j: (0, 0)),
            pl.BlockSpec((K, 1), lambda p, j: (0, 0)),
            pl.BlockSpec((K, 1), lambda p, j: (0, 0)),
            pl.BlockSpec((K // 2, K), lambda p, j: (0, 0)),
            pl.BlockSpec((K // 2, 1), lambda p, j: (0, 0)),
            pl.BlockSpec((K, K // 2), lambda p, j: (0, 0)),
            pl.BlockSpec((K, 1), lambda p, j: (0, 0)),
            pl.BlockSpec((D, OUT), lambda p, j: (0, 0)),
            pl.BlockSpec((1, OUT), lambda p, j: (0, 0)),
        ],
        out_specs=pl.BlockSpec((TN, OUT), lambda p, j: (p * j, 0)),
        out_shape=jax.ShapeDtypeStruct((N, OUT), jnp.float32),
        scratch_shapes=[pltpu.VMEM((K, D), jnp.float32),
                        pltpu.VMEM((K, OUT), jnp.float32)],
    )(x, evt, evals_col, mask_col, w1t, b1_col, w2t, b2_col, Wp, bp_row)
    return out


# speedup vs baseline: 1.0206x; 1.0206x over previous
import jax
import jax.numpy as jnp
from jax.experimental import pallas as pl
from jax.experimental.pallas import tpu as pltpu

N = 100000
D = 128
K = 16
OUT = 256
TN = 12800
NT = (N + TN - 1) // TN
TAIL = N - (NT - 1) * TN


def _body(x_ref, evt_ref, evals_ref, mask_ref, w1t_ref, b1_ref, w2t_ref,
          b2_ref, wp_ref, bp_ref, out_ref, acc_ref, m_ref):
    p = pl.program_id(0)
    j = pl.program_id(1)

    @pl.when(jnp.logical_and(p == 0, j == 0))
    def _():
        acc_ref[...] = jax.lax.dot_general(
            evt_ref[:, pl.ds(0, TN)], x_ref[...],
            dimension_numbers=(((1,), (0,)), ((), ())),
            preferred_element_type=jnp.float32)

    @pl.when(jnp.logical_and(p == 0,
                             jnp.logical_and(j > 0, j < NT - 1)))
    def _():
        evt = evt_ref[:, pl.ds(j * TN, TN)]
        acc_ref[...] += jax.lax.dot_general(
            evt, x_ref[...],
            dimension_numbers=(((1,), (0,)), ((), ())),
            preferred_element_type=jnp.float32)

    @pl.when(jnp.logical_and(p == 0, j == NT - 1))
    def _():
        evt = evt_ref[:, pl.ds((NT - 1) * TN, TAIL)]
        xfreq = acc_ref[...] + jax.lax.dot_general(
            evt, x_ref[0:TAIL, :],
            dimension_numbers=(((1,), (0,)), ((), ())),
            preferred_element_type=jnp.float32)
        h = jnp.maximum(
            jnp.dot(w1t_ref[...], evals_ref[...],
                    preferred_element_type=jnp.float32) + b1_ref[...], 0.0)
        filt = jnp.tanh(
            jnp.dot(w2t_ref[...], h,
                    preferred_element_type=jnp.float32) + b2_ref[...])
        filt = filt * mask_ref[...]
        m_ref[...] = jnp.dot(filt * xfreq, wp_ref[...],
                             preferred_element_type=jnp.float32)

    @pl.when(jnp.logical_and(p == 1, j < NT - 1))
    def _():
        evt = evt_ref[:, pl.ds(j * TN, TN)]
        out_ref[...] = jax.lax.dot_general(
            evt, m_ref[...],
            dimension_numbers=(((0,), (0,)), ((), ())),
            preferred_element_type=jnp.float32) + bp_ref[...]

    @pl.when(jnp.logical_and(p == 1, j == NT - 1))
    def _():
        evt = evt_ref[:, pl.ds((NT - 1) * TN, TAIL)]
        out_ref[0:TAIL, :] = jax.lax.dot_general(
            evt, m_ref[...],
            dimension_numbers=(((0,), (0,)), ((), ())),
            preferred_element_type=jnp.float32) + bp_ref[...]


def kernel(x, eigvecs, eigvals, eig_mask, W1, b1, W2, b2, Wp, bp):
    evt = eigvecs.T
    evals_col = eigvals.reshape(K, 1)
    mask_col = eig_mask.astype(jnp.float32).reshape(K, 1)
    w1t = W1.T
    b1_col = b1.reshape(K // 2, 1)
    w2t = W2.T
    b2_col = b2.reshape(K, 1)
    bp_row = bp.reshape(1, OUT)

    out = pl.pallas_call(
        _body,
        grid=(2, NT),
        in_specs=[
            pl.BlockSpec((TN, D), lambda p, j: ((1 - p) * j + p * (NT - 1), 0)),
            pl.BlockSpec((K, N), lambda p, j: (0, 0)),
            pl.BlockSpec((K, 1), lambda p, j: (0, 0)),
            pl.BlockSpec((K, 1), lambda p, j: (0, 0)),
            pl.BlockSpec((K // 2, K), lambda p, j: (0, 0)),
            pl.BlockSpec((K // 2, 1), lambda p, j: (0, 0)),
            pl.BlockSpec((K, K // 2), lambda p, j: (0, 0)),
            pl.BlockSpec((K, 1), lambda p, j: (0, 0)),
            pl.BlockSpec((D, OUT), lambda p, j: (0, 0)),
            pl.BlockSpec((1, OUT), lambda p, j: (0, 0)),
        ],
        out_specs=pl.BlockSpec((TN, OUT), lambda p, j: (p * j, 0)),
        out_shape=jax.ShapeDtypeStruct((N, OUT), jnp.float32),
        scratch_shapes=[pltpu.VMEM((K, D), jnp.float32),
                        pltpu.VMEM((K, OUT), jnp.float32)],
    )(x, evt, evals_col, mask_col, w1t, b1_col, w2t, b2_col, Wp, bp_row)
    return out
